# baseline (speedup 1.0000x reference)
import math

import jax
import jax.numpy as jnp
from jax.experimental import pallas as pl
from jax.experimental.pallas import tpu as pltpu

_SIZE = 32000
_PAD = 0
_SMOOTH = 0.1
_CONF = 1.0 - _SMOOTH
_EPS = _SMOOTH / (_SIZE - 1)
_C = (_SIZE - 1) * _EPS * math.log(_EPS) + _CONF * math.log(_CONF)

_R = 64
_CB = 32000


def _ls_kernel(tgt_ref, x_ref, out_ref, acc_ref, tok_ref):
    i = pl.program_id(0)
    ni = pl.num_programs(0)

    @pl.when(i == 0)
    def _init():
        acc_ref[0, 0] = 0.0
        tok_ref[0, 0] = 0.0

    x = x_ref[...]
    tgt = tgt_ref[0]
    tgt_col = tgt.reshape(_R, 1)
    maskv = tgt_col != _PAD

    rowsum = jnp.sum(x, axis=1, keepdims=True)
    col = jax.lax.broadcasted_iota(jnp.int32, (_R, _CB), 1)
    xt = jnp.sum(jnp.where(col == tgt_col, x, 0.0), axis=1, keepdims=True)
    contrib = jnp.where(maskv, -_EPS * rowsum - (_CONF - _EPS) * xt, 0.0)
    mask_cnt = jnp.sum(maskv.astype(jnp.float32))
    acc_ref[0, 0] += jnp.sum(contrib) + _C * mask_cnt
    tok_ref[0, 0] += mask_cnt

    @pl.when(i == ni - 1)
    def _finish():
        out_ref[0, 0] = acc_ref[0, 0] / tok_ref[0, 0]


def kernel(x, target):
    n = x.shape[0]
    g = n // _R
    tgt_blocks = target.astype(jnp.int32).reshape(g, 1, _R)
    out = pl.pallas_call(
        _ls_kernel,
        grid=(g,),
        in_specs=[
            pl.BlockSpec((1, 1, _R), lambda i: (i, 0, 0)),
            pl.BlockSpec((_R, _CB), lambda i: (i, 0)),
        ],
        out_specs=pl.BlockSpec(memory_space=pltpu.SMEM),
        out_shape=jax.ShapeDtypeStruct((1, 1), jnp.float32),
        scratch_shapes=[
            pltpu.SMEM((1, 1), jnp.float32),
            pltpu.SMEM((1, 1), jnp.float32),
        ],
    )(tgt_blocks, x)
    return out[0, 0]


# speedup vs baseline: 1.0674x; 1.0674x over previous
import math

import jax
import jax.numpy as jnp
from jax.experimental import pallas as pl
from jax.experimental.pallas import tpu as pltpu

_SIZE = 32000
_PAD = 0
_SMOOTH = 0.1
_CONF = 1.0 - _SMOOTH
_EPS = _SMOOTH / (_SIZE - 1)
_C = (_SIZE - 1) * _EPS * math.log(_EPS) + _CONF * math.log(_CONF)

_R = 256
_CB = 32000


def _ls_kernel(tgt_ref, x_ref, out_ref, acc_ref, tok_ref):
    i = pl.program_id(0)
    ni = pl.num_programs(0)

    @pl.when(i == 0)
    def _init():
        acc_ref[0, 0] = 0.0
        tok_ref[0, 0] = 0.0

    x = x_ref[...]
    tgt = tgt_ref[0]
    tgt_col = tgt.reshape(_R, 1)
    maskv = tgt_col != _PAD

    rowsum = jnp.sum(x, axis=1, keepdims=True)
    col = jax.lax.broadcasted_iota(jnp.int32, (_R, _CB), 1)
    xt = jnp.sum(jnp.where(col == tgt_col, x, 0.0), axis=1, keepdims=True)
    contrib = jnp.where(maskv, -_EPS * rowsum - (_CONF - _EPS) * xt, 0.0)
    mask_cnt = jnp.sum(maskv.astype(jnp.float32))
    acc_ref[0, 0] += jnp.sum(contrib) + _C * mask_cnt
    tok_ref[0, 0] += mask_cnt

    @pl.when(i == ni - 1)
    def _finish():
        out_ref[0, 0] = acc_ref[0, 0] / tok_ref[0, 0]


def kernel(x, target):
    n = x.shape[0]
    g = n // _R
    tgt_blocks = target.astype(jnp.int32).reshape(g, 1, _R)
    out = pl.pallas_call(
        _ls_kernel,
        grid=(g,),
        in_specs=[
            pl.BlockSpec((1, 1, _R), lambda i: (i, 0, 0)),
            pl.BlockSpec((_R, _CB), lambda i: (i, 0)),
        ],
        out_specs=pl.BlockSpec(memory_space=pltpu.SMEM),
        out_shape=jax.ShapeDtypeStruct((1, 1), jnp.float32),
        scratch_shapes=[
            pltpu.SMEM((1, 1), jnp.float32),
            pltpu.SMEM((1, 1), jnp.float32),
        ],
        compiler_params=pltpu.CompilerParams(vmem_limit_bytes=100 * 1024 * 1024),
    )(tgt_blocks, x)
    return out[0, 0]


# speedup vs baseline: 1.0856x; 1.0171x over previous
import math

import jax
import jax.numpy as jnp
from jax.experimental import pallas as pl
from jax.experimental.pallas import tpu as pltpu

_SIZE = 32000
_PAD = 0
_SMOOTH = 0.1
_CONF = 1.0 - _SMOOTH
_EPS = _SMOOTH / (_SIZE - 1)
_C = (_SIZE - 1) * _EPS * math.log(_EPS) + _CONF * math.log(_CONF)

_R = 128
_CB = 32000


def _ls_kernel(tgt_ref, x_ref, out_ref, acc_ref, tok_ref):
    i = pl.program_id(0)
    ni = pl.num_programs(0)

    @pl.when(i == 0)
    def _init():
        acc_ref[0, 0] = 0.0
        tok_ref[0, 0] = 0.0

    x = x_ref[...]
    tgt = tgt_ref[0]
    tgt_col = tgt.reshape(_R, 1)
    maskv = tgt_col != _PAD

    rowsum = jnp.sum(x, axis=1, keepdims=True)
    col = jax.lax.broadcasted_iota(jnp.int32, (_R, _CB), 1)
    xt = jnp.sum(jnp.where(col == tgt_col, x, 0.0), axis=1, keepdims=True)
    contrib = jnp.where(maskv, -_EPS * rowsum - (_CONF - _EPS) * xt, 0.0)
    mask_cnt = jnp.sum(maskv.astype(jnp.float32))
    acc_ref[0, 0] += jnp.sum(contrib) + _C * mask_cnt
    tok_ref[0, 0] += mask_cnt

    @pl.when(i == ni - 1)
    def _finish():
        out_ref[0, 0] = acc_ref[0, 0] / tok_ref[0, 0]


def kernel(x, target):
    n = x.shape[0]
    g = n // _R
    tgt_blocks = target.astype(jnp.int32).reshape(g, 1, _R)
    out = pl.pallas_call(
        _ls_kernel,
        grid=(g,),
        in_specs=[
            pl.BlockSpec((1, 1, _R), lambda i: (i, 0, 0)),
            pl.BlockSpec((_R, _CB), lambda i: (i, 0)),
        ],
        out_specs=pl.BlockSpec(memory_space=pltpu.SMEM),
        out_shape=jax.ShapeDtypeStruct((1, 1), jnp.float32),
        scratch_shapes=[
            pltpu.SMEM((1, 1), jnp.float32),
            pltpu.SMEM((1, 1), jnp.float32),
        ],
    )(tgt_blocks, x)
    return out[0, 0]
